# Initial kernel scaffold; baseline (speedup 1.0000x reference)
import jax
import jax.numpy as jnp
from jax.experimental import pallas as pl


def kernel(shape_features, faces, W1, b1, W2, b2):
    raise NotImplementedError("write your pallas kernel here")


# speedup vs baseline: 22.7819x; 22.7819x over previous
import functools

import jax
import jax.numpy as jnp
from jax import lax
from jax.experimental import pallas as pl
from jax.experimental.pallas import tpu as pltpu
from jax.experimental.pallas import tpu_sc as plsc

N = 10000
D = 256
NTILES = 32
VPT = 313
ACC_ROWS = 320
TRASH = 316
B_BLK = 128
MAX_BLKS = 960
ZROW = N


def _extract(vec, lane):
    onehot = (lax.iota(jnp.int32, 16) == lane).astype(jnp.int32)
    return jnp.sum(vec * onehot)


def _sc_aggregate(x_pad, dj, dl, blk0, blk1, zin):
    mesh = plsc.VectorSubcoreMesh(core_axis_name="c", subcore_axis_name="s")

    @functools.partial(
        pl.kernel,
        mesh=mesh,
        compiler_params=pltpu.CompilerParams(needs_layout_passes=False),
        out_type=jax.ShapeDtypeStruct((NTILES, ACC_ROWS * D), jnp.float32),
        scratch_types=[
            pltpu.VMEM((B_BLK,), jnp.int32),
            pltpu.VMEM((B_BLK,), jnp.int32),
            pltpu.VMEM((B_BLK, D), jnp.float32),
            pltpu.VMEM((ACC_ROWS * D,), jnp.float32),
            pltpu.VMEM((16,), jnp.int32),
            pltpu.VMEM((16,), jnp.int32),
            pltpu.SemaphoreType.DMA,
        ],
    )
    def k(x_hbm, dj_hbm, dl_hbm, b0_hbm, b1_hbm, z_hbm, out_hbm,
          djv, dlv, rows, acc, b0v, b1v, sem):
        c = lax.axis_index("c")
        s = lax.axis_index("s")
        w = s * 2 + c

        half = (w >= 16).astype(jnp.int32)
        lane = w - half * 16
        pltpu.sync_copy(b0_hbm.at[pl.ds(half * 16, 16)], b0v)
        pltpu.sync_copy(b1_hbm.at[pl.ds(half * 16, 16)], b1v)
        pltpu.sync_copy(z_hbm, acc)
        my_b0 = _extract(b0v[...], lane)
        my_b1 = _extract(b1v[...], lane)
        dbase = w * VPT

        def block(bi, carry):
            b = my_b0 + bi

            @pl.when(b < my_b1)
            def _():
                base = b * B_BLK
                pltpu.sync_copy(dj_hbm.at[pl.ds(base, B_BLK)], djv)
                pltpu.sync_copy(dl_hbm.at[pl.ds(base, B_BLK)], dlv)
                pltpu.async_copy(x_hbm.at[djv], rows, sem).wait()

                def group(g, carry2):
                    dl16 = dlv[pl.ds(g * 16, 16)]
                    for l in range(16):
                        d = _extract(dl16, jnp.int32(l))
                        loc = d - dbase
                        ok = (loc >= 0) & (loc < VPT)
                        tgt = lax.select(ok, loc, jnp.int32(TRASH))
                        abase = tgt * D
                        r = g * 16 + l
                        for cc in range(16):
                            asl = pl.ds(abase + cc * 16, 16)
                            sl = pl.ds(cc * 16, 16)
                            acc[asl] = acc[asl] + rows[r, sl]
                    return carry2

                lax.fori_loop(0, B_BLK // 16, group, 0)

            return carry

        lax.fori_loop(0, MAX_BLKS, block, 0)
        pltpu.sync_copy(acc, out_hbm.at[w])

    return k(x_pad, dj, dl, blk0, blk1, zin)


def _tc_body(x_ref, aggr_ref, w1_ref, w2_ref, bias_ref, out_ref):
    dn = (((1,), (1,)), ((), ()))
    out_ref[...] = (
        lax.dot_general(x_ref[...], w1_ref[...], dn,
                        preferred_element_type=jnp.float32)
        + lax.dot_general(aggr_ref[...], w2_ref[...], dn,
                          preferred_element_type=jnp.float32)
        + bias_ref[...]
    )


def _tc_linear(x, aggr, W1, W2, bias):
    rows_blk = 1000
    grid = (N // rows_blk,)
    return pl.pallas_call(
        _tc_body,
        grid=grid,
        in_specs=[
            pl.BlockSpec((rows_blk, D), lambda i: (i, 0)),
            pl.BlockSpec((rows_blk, D), lambda i: (i, 0)),
            pl.BlockSpec((D, D), lambda i: (0, 0)),
            pl.BlockSpec((D, D), lambda i: (0, 0)),
            pl.BlockSpec((1, D), lambda i: (0, 0)),
        ],
        out_specs=pl.BlockSpec((rows_blk, D), lambda i: (i, 0)),
        out_shape=jax.ShapeDtypeStruct((N, D), jnp.float32),
    )(x, aggr, W1, W2, bias)


def kernel(shape_features, faces, W1, b1, W2, b2):
    f0, f1, f2 = faces[:, 0], faces[:, 1], faces[:, 2]
    src = jnp.concatenate([f0, f0, f1, f1, f2, f2])
    dst = jnp.concatenate([f1, f2, f0, f2, f0, f1])
    key = src * N + dst
    skey = jnp.sort(key)
    valid = jnp.concatenate(
        [jnp.ones((1,), jnp.bool_), skey[1:] != skey[:-1]])
    di = skey // N
    dj = jnp.where(valid, skey % N, ZROW).astype(jnp.int32)

    e = key.shape[0]
    e_pad = ((e + B_BLK - 1) // B_BLK) * B_BLK
    pad = e_pad - e
    dj = jnp.concatenate([dj, jnp.full((pad,), ZROW, jnp.int32)])
    dl = jnp.concatenate([di.astype(jnp.int32),
                          jnp.full((pad,), 1 << 20, jnp.int32)])

    bounds = jnp.arange(NTILES + 1, dtype=jnp.int32) * VPT
    lo = jnp.searchsorted(di, bounds[:-1], side="left").astype(jnp.int32)
    hi = jnp.searchsorted(di, bounds[1:], side="left").astype(jnp.int32)
    blk0 = lo // B_BLK
    blk1 = (hi + B_BLK - 1) // B_BLK

    x_pad = jnp.concatenate(
        [shape_features, jnp.zeros((8, D), jnp.float32)], axis=0)
    zin = jnp.zeros((ACC_ROWS * D,), jnp.float32)

    out = _sc_aggregate(x_pad, dj, dl, blk0, blk1, zin)
    aggr = out.reshape(NTILES, ACC_ROWS, D)[:, :VPT, :].reshape(
        NTILES * VPT, D)[:N]

    bias = (b1 + b2).reshape(1, D)
    return _tc_linear(shape_features, aggr, W1, W2, bias)


# speedup vs baseline: 26.3436x; 1.1563x over previous
import functools

import jax
import jax.numpy as jnp
from jax import lax
from jax.experimental import pallas as pl
from jax.experimental.pallas import tpu as pltpu
from jax.experimental.pallas import tpu_sc as plsc

N = 10000
D = 256
NTILES = 32
VPT = 313
ACC_ROWS = 320
TRASH = 316
B_BLK = 64
BW = 64
PF = B_BLK * BW
MAX_WIN = 30
ZROW = N


def _extract(vec, lane):
    onehot = (lax.iota(jnp.int32, 16) == lane).astype(jnp.int32)
    return jnp.sum(vec * onehot)


def _sc_aggregate(x_pad, dj, dl, blk0, blk1, zin):
    mesh = plsc.VectorSubcoreMesh(core_axis_name="c", subcore_axis_name="s")

    @functools.partial(
        pl.kernel,
        mesh=mesh,
        compiler_params=pltpu.CompilerParams(needs_layout_passes=False),
        out_type=jax.ShapeDtypeStruct((NTILES, ACC_ROWS * D), jnp.float32),
        scratch_types=[
            pltpu.VMEM((PF,), jnp.int32),
            pltpu.VMEM((PF,), jnp.int32),
            pltpu.VMEM((B_BLK, D), jnp.float32),
            pltpu.VMEM((B_BLK, D), jnp.float32),
            pltpu.VMEM((ACC_ROWS * D,), jnp.float32),
            pltpu.VMEM((16,), jnp.int32),
            pltpu.VMEM((16,), jnp.int32),
            pltpu.SemaphoreType.DMA,
            pltpu.SemaphoreType.DMA,
        ],
    )
    def k(x_hbm, dj_hbm, dl_hbm, b0_hbm, b1_hbm, z_hbm, out_hbm,
          djall, dlall, rows0, rows1, acc, b0v, b1v, sem0, sem1):
        c = lax.axis_index("c")
        s = lax.axis_index("s")
        w = s * 2 + c

        half = (w >= 16).astype(jnp.int32)
        lane = w - half * 16
        pltpu.sync_copy(b0_hbm.at[pl.ds(half * 16, 16)], b0v)
        pltpu.sync_copy(b1_hbm.at[pl.ds(half * 16, 16)], b1v)
        pltpu.sync_copy(z_hbm, acc)
        my_b0 = _extract(b0v[...], lane)
        my_b1 = _extract(b1v[...], lane)
        dbase = w * VPT

        rows_bufs = (rows0, rows1)
        sems = (sem0, sem1)

        def issue(kbuf, off):
            pltpu.async_copy(x_hbm.at[djall.at[pl.ds(off, B_BLK)]],
                             rows_bufs[kbuf], sems[kbuf])

        def drain(kbuf):
            pltpu.make_async_copy(x_hbm.at[pl.ds(0, B_BLK)],
                                  rows_bufs[kbuf], sems[kbuf]).wait()

        def compute(kbuf, eoff):
            rows = rows_bufs[kbuf]

            def group(g, carry2):
                dl16 = dlall[pl.ds(eoff + g * 16, 16)]
                for l in range(16):
                    d = dl16[l]
                    loc = d - dbase
                    ok = (loc >= 0) & (loc < VPT)
                    tgt = lax.select(ok, loc, jnp.int32(TRASH))
                    abase = tgt * D
                    r = g * 16 + l
                    for cc in range(16):
                        asl = pl.ds(abase + cc * 16, 16)
                        sl = pl.ds(cc * 16, 16)
                        acc[asl] = acc[asl] + rows[r, sl]
                return carry2

            lax.fori_loop(0, B_BLK // 16, group, 0)

        def window(win, carry):
            w0 = my_b0 + win * BW

            @pl.when(w0 < my_b1)
            def _():
                e0 = w0 * B_BLK
                pltpu.sync_copy(dj_hbm.at[pl.ds(e0, PF)], djall)
                pltpu.sync_copy(dl_hbm.at[pl.ds(e0, PF)], dlall)
                issue(0, 0)

                def pair(pi, c2):
                    b_a = w0 + 2 * pi
                    b_b = b_a + 1

                    @pl.when(b_b < my_b1)
                    def _():
                        issue(1, (2 * pi + 1) * B_BLK)

                    @pl.when(b_a < my_b1)
                    def _():
                        drain(0)
                        compute(0, 2 * pi * B_BLK)

                    @pl.when((b_a + 2 < my_b1) & (2 * pi + 2 < BW))
                    def _():
                        issue(0, (2 * pi + 2) * B_BLK)

                    @pl.when(b_b < my_b1)
                    def _():
                        drain(1)
                        compute(1, (2 * pi + 1) * B_BLK)

                    return c2

                lax.fori_loop(0, BW // 2, pair, 0)

            return carry

        lax.fori_loop(0, MAX_WIN, window, 0)
        pltpu.sync_copy(acc, out_hbm.at[w])

    return k(x_pad, dj, dl, blk0, blk1, zin)


def _tc_body(x_ref, aggr_ref, w1_ref, w2_ref, bias_ref, out_ref):
    dn = (((1,), (1,)), ((), ()))
    out_ref[...] = (
        lax.dot_general(x_ref[...], w1_ref[...], dn,
                        preferred_element_type=jnp.float32)
        + lax.dot_general(aggr_ref[...], w2_ref[...], dn,
                          preferred_element_type=jnp.float32)
        + bias_ref[...]
    )


def _tc_linear(x, aggr, W1, W2, bias):
    rows_blk = 1000
    return pl.pallas_call(
        _tc_body,
        grid=(N // rows_blk,),
        in_specs=[
            pl.BlockSpec((rows_blk, D), lambda i: (i, 0)),
            pl.BlockSpec((rows_blk, D), lambda i: (i, 0)),
            pl.BlockSpec((D, D), lambda i: (0, 0)),
            pl.BlockSpec((D, D), lambda i: (0, 0)),
            pl.BlockSpec((1, D), lambda i: (0, 0)),
        ],
        out_specs=pl.BlockSpec((rows_blk, D), lambda i: (i, 0)),
        out_shape=jax.ShapeDtypeStruct((N, D), jnp.float32),
    )(x, aggr, W1, W2, bias)


def kernel(shape_features, faces, W1, b1, W2, b2):
    f0, f1, f2 = faces[:, 0], faces[:, 1], faces[:, 2]
    src = jnp.concatenate([f0, f0, f1, f1, f2, f2])
    dst = jnp.concatenate([f1, f2, f0, f2, f0, f1])
    key = src * N + dst
    skey = jnp.sort(key)
    valid = jnp.concatenate(
        [jnp.ones((1,), jnp.bool_), skey[1:] != skey[:-1]])
    di = skey // N
    dj = jnp.where(valid, skey % N, ZROW).astype(jnp.int32)

    e = key.shape[0]
    e_pad = ((e + B_BLK - 1) // B_BLK) * B_BLK
    pad = e_pad - e + PF
    dj = jnp.concatenate([dj, jnp.full((pad,), ZROW, jnp.int32)])
    dl = jnp.concatenate([di.astype(jnp.int32),
                          jnp.full((pad,), 1 << 20, jnp.int32)])

    bounds = jnp.arange(NTILES + 1, dtype=jnp.int32) * VPT
    lo = jnp.searchsorted(di, bounds[:-1], side="left").astype(jnp.int32)
    hi = jnp.searchsorted(di, bounds[1:], side="left").astype(jnp.int32)
    blk0 = lo // B_BLK
    blk1 = (hi + B_BLK - 1) // B_BLK

    x_pad = jnp.concatenate(
        [shape_features, jnp.zeros((8, D), jnp.float32)], axis=0)
    zin = jnp.zeros((ACC_ROWS * D,), jnp.float32)

    out = _sc_aggregate(x_pad, dj, dl, blk0, blk1, zin)
    aggr = out.reshape(NTILES, ACC_ROWS, D)[:, :VPT, :].reshape(
        NTILES * VPT, D)[:N]

    bias = (b1 + b2).reshape(1, D)
    return _tc_linear(shape_features, aggr, W1, W2, bias)
